# Initial kernel scaffold; baseline (speedup 1.0000x reference)
import functools

import jax
import jax.numpy as jnp
from jax import lax
from jax.experimental import pallas as pl
from jax.experimental.pallas import tpu as pltpu
from jax.experimental.pallas import tpu_sc as plsc

NC = 2
NS = 16
NW = NC * NS
LANES = 16
C = 128
NCH = 11
NKEEP = 3
D = 64


def _body(n_chunks, x_hbm, tab_hbm, xo_hbm, ebd_hbm, xbuf, idx_v, rows_v,
          xo_v, sem):
    wid = lax.axis_index("s") * NC + lax.axis_index("c")
    row0 = wid * (n_chunks * C)

    lanes = lax.iota(jnp.int32, LANES)

    def chunk(g, carry):
        base = row0 + g * C
        pltpu.sync_copy(x_hbm.at[pl.ds(base * NCH, C * NCH)], xbuf)

        for j in range(C // LANES):
            src = (lanes + j * LANES) * NCH + 3
            v = plsc.load_gather(xbuf, [src])
            idx_v[pl.ds(j * LANES, LANES)] = (v * 288.0).astype(jnp.int32)

        pltpu.async_copy(tab_hbm.at[idx_v], rows_v, sem).wait()
        pltpu.sync_copy(rows_v, ebd_hbm.at[pl.ds(base, C)])

        for j in range(C * NKEEP // LANES):
            o = lanes + j * LANES
            r = o // NKEEP
            src = r * NCH + (o - r * NKEEP)
            xo_v[pl.ds(j * LANES, LANES)] = plsc.load_gather(xbuf, [src])
        pltpu.sync_copy(xo_v, xo_hbm.at[pl.ds(base * NKEEP, C * NKEEP)])
        return carry

    lax.fori_loop(0, n_chunks, chunk, 0)


@jax.jit
def kernel(x, time_table):
    b, n, t, ch = x.shape
    m = b * n * t
    assert ch == NCH and m % (NW * C) == 0
    n_chunks = m // (NW * C)

    mesh = plsc.VectorSubcoreMesh(core_axis_name="c", subcore_axis_name="s")
    xo_flat, ebd = pl.kernel(
        functools.partial(_body, n_chunks),
        out_type=(
            jax.ShapeDtypeStruct((m * NKEEP,), jnp.float32),
            jax.ShapeDtypeStruct((m, D), jnp.float32),
        ),
        mesh=mesh,
        scratch_types=[
            pltpu.VMEM((C * NCH,), jnp.float32),
            pltpu.VMEM((C,), jnp.int32),
            pltpu.VMEM((C, D), jnp.float32),
            pltpu.VMEM((C * NKEEP,), jnp.float32),
            pltpu.SemaphoreType.DMA,
        ],
    )(x.reshape(-1), time_table)
    return xo_flat.reshape(b, n, t, NKEEP), ebd.reshape(b, n, t, D)


# speedup vs baseline: 1.8383x; 1.8383x over previous
import functools

import jax
import jax.numpy as jnp
from jax import lax
from jax.experimental import pallas as pl
from jax.experimental.pallas import tpu as pltpu
from jax.experimental.pallas import tpu_sc as plsc

NC = 2
NS = 16
NW = NC * NS
LANES = 16
C = 128
NCH = 11
NKEEP = 3
D = 64


def _body(n_chunks, x_hbm, tab_hbm, xo_hbm, ebd_hbm, xbuf, idx_v, rows_v,
          xo_v, sem):
    wid = lax.axis_index("s") * NC + lax.axis_index("c")
    row0 = wid * (n_chunks * C)

    lanes = lax.iota(jnp.int32, LANES)

    def chunk(g, carry):
        base = row0 + g * C
        pltpu.sync_copy(x_hbm.at[pl.ds(base * NCH, C * NCH)], xbuf)

        for j in range(C // LANES):
            src = (lanes + j * LANES) * NCH + 3
            v = plsc.load_gather(xbuf, [src])
            idx_v[pl.ds(j * LANES, LANES)] = (v * 288.0).astype(jnp.int32)

        pltpu.async_copy(tab_hbm.at[idx_v], rows_v, sem).wait()
        pltpu.sync_copy(rows_v, ebd_hbm.at[pl.ds(base, C)])

        for j in range(C * NKEEP // LANES):
            o = lanes + j * LANES
            r = o // NKEEP
            src = r * NCH + (o - r * NKEEP)
            xo_v[pl.ds(j * LANES, LANES)] = plsc.load_gather(xbuf, [src])
        pltpu.sync_copy(xo_v, xo_hbm.at[pl.ds(base * NKEEP, C * NKEEP)])
        return carry

    lax.fori_loop(0, n_chunks, chunk, 0)


@jax.jit
def kernel(x, time_table):
    b, n, t, ch = x.shape
    m = b * n * t
    assert ch == NCH and m % (NW * C) == 0
    n_chunks = m // (NW * C)

    mesh = plsc.VectorSubcoreMesh(core_axis_name="c", subcore_axis_name="s")
    xo_flat, ebd = pl.kernel(
        functools.partial(_body, n_chunks),
        out_type=(
            jax.ShapeDtypeStruct((m * NKEEP,), jnp.float32),
            jax.ShapeDtypeStruct((m, D), jnp.float32),
        ),
        mesh=mesh,
        compiler_params=pltpu.CompilerParams(
            needs_layout_passes=False, use_tc_tiling_on_sc=False),
        scratch_types=[
            pltpu.VMEM((C * NCH,), jnp.float32),
            pltpu.VMEM((C,), jnp.int32),
            pltpu.VMEM((C, D), jnp.float32),
            pltpu.VMEM((C * NKEEP,), jnp.float32),
            pltpu.SemaphoreType.DMA,
        ],
    )(x.reshape(-1), time_table)
    return xo_flat.reshape(b, n, t, NKEEP), ebd.reshape(b, n, t, D)


# speedup vs baseline: 2.5967x; 1.4126x over previous
import functools

import jax
import jax.numpy as jnp
from jax import lax
from jax.experimental import pallas as pl
from jax.experimental.pallas import tpu as pltpu
from jax.experimental.pallas import tpu_sc as plsc

NC = 2
NS = 16
NW = NC * NS
LANES = 16
C = 128
NCH = 11
NKEEP = 3
NT = 288
D = 64
RTC = 4096


def _sc_body(n_chunks, x_hbm, tab_hbm, ebd_hbm, tabv, xbuf, ebuf):
    wid = lax.axis_index("s") * NC + lax.axis_index("c")
    row0 = wid * (n_chunks * C)

    pltpu.sync_copy(tab_hbm, tabv)

    lanes = lax.iota(jnp.int32, LANES)
    col3 = lanes * 0 + 3

    def chunk(g, carry):
        base = row0 + g * C
        pltpu.sync_copy(x_hbm.at[pl.ds(base, C), :], xbuf)

        for j in range(C // LANES):
            v = plsc.load_gather(xbuf, [lanes + j * LANES, col3])
            toff = (v * 288.0).astype(jnp.int32) * D
            for l in range(LANES):
                t = toff[l]
                rr = j * LANES + l
                for k in range(D // LANES):
                    ebuf[rr, pl.ds(k * LANES, LANES)] = tabv[
                        pl.ds(t + k * LANES, LANES)]

        pltpu.sync_copy(ebuf, ebd_hbm.at[pl.ds(base, C), :])
        return carry

    lax.fori_loop(0, n_chunks, chunk, 0)


def _tc_slice_body(x_ref, o_ref):
    o_ref[...] = x_ref[:, :NKEEP]


@jax.jit
def kernel(x, time_table):
    b, n, t, ch = x.shape
    m = b * n * t
    assert ch == NCH and m % (NW * C) == 0 and m % RTC == 0
    n_chunks = m // (NW * C)
    x2d = x.reshape(m, NCH)

    mesh = plsc.VectorSubcoreMesh(core_axis_name="c", subcore_axis_name="s")
    ebd = pl.kernel(
        functools.partial(_sc_body, n_chunks),
        out_type=jax.ShapeDtypeStruct((m, D), jnp.float32),
        mesh=mesh,
        compiler_params=pltpu.CompilerParams(
            needs_layout_passes=False, use_tc_tiling_on_sc=True),
        scratch_types=[
            pltpu.VMEM((NT * D,), jnp.float32),
            pltpu.VMEM((C, NCH), jnp.float32),
            pltpu.VMEM((C, D), jnp.float32),
        ],
    )(x2d, time_table.reshape(NT * D))

    xo = pl.pallas_call(
        _tc_slice_body,
        grid=(m // RTC,),
        in_specs=[pl.BlockSpec((RTC, NCH), lambda i: (i, 0))],
        out_specs=pl.BlockSpec((RTC, NKEEP), lambda i: (i, 0)),
        out_shape=jax.ShapeDtypeStruct((m, NKEEP), jnp.float32),
    )(x2d)

    return xo.reshape(b, n, t, NKEEP), ebd.reshape(b, n, t, D)


# speedup vs baseline: 22.9279x; 8.8296x over previous
import jax
import jax.numpy as jnp
from jax import lax
from jax.experimental import pallas as pl
from jax.experimental.pallas import tpu as pltpu
from jax.experimental.pallas import tpu_sc as plsc

NC = 2
NS = 16
NW = NC * NS
LANES = 16
NCH = 11
NKEEP = 3
NT = 288
D = 64
N = 512
B = 16
G8 = 8
BPW = B * NT // NW


def _sc_body(x_rows, tab_hbm, ebd_hbm, tabv, xbuf, ebuf0, ebuf1, sem0, sem1):
    wid = lax.axis_index("s") * NC + lax.axis_index("c")
    b = wid >> 1
    t0 = (wid & 1) * BPW

    pltpu.sync_copy(tab_hbm, tabv)

    xrow0 = b * (NCH * NT) + 3 * NT + t0
    bt0 = b * NT + t0

    def i_body(i, carry):
        pltpu.sync_copy(x_rows.at[pl.ds(xrow0 + i * G8, G8), :], xbuf)
        for r in range(G8):
            ebuf = ebuf0 if r % 2 == 0 else ebuf1
            sem = sem0 if r % 2 == 0 else sem1

            if r >= 2:
                pltpu.make_async_copy(ebuf, ebd_hbm.at[pl.ds(0, D), :], sem).wait()
            else:
                @pl.when(i > 0)
                def _():
                    pltpu.make_async_copy(
                        ebuf, ebd_hbm.at[pl.ds(0, D), :], sem).wait()

            @plsc.parallel_loop(0, N // LANES, step=1)
            def j_loop(j):
                v = xbuf[r, pl.ds(j * LANES, LANES)]
                base = (v * 288.0).astype(jnp.int32)
                for d in range(D):
                    g = plsc.load_gather(tabv, [base + d * NT])
                    ebuf[d, pl.ds(j * LANES, LANES)] = g

            out_row = (bt0 + i * G8 + r) * D
            pltpu.async_copy(ebuf, ebd_hbm.at[pl.ds(out_row, D), :], sem)
        return carry

    lax.fori_loop(0, BPW // G8, i_body, 0)
    pltpu.make_async_copy(ebuf0, ebd_hbm.at[pl.ds(0, D), :], sem0).wait()
    pltpu.make_async_copy(ebuf1, ebd_hbm.at[pl.ds(0, D), :], sem1).wait()


def _tc_slice_body(x_ref, o_ref):
    o_ref[...] = x_ref[...]


@jax.jit
def kernel(x, time_table):
    b, n, t, ch = x.shape
    assert (b, n, t, ch) == (B, N, NT, NCH) and time_table.shape == (NT, D)

    xt = jnp.transpose(x, (0, 3, 2, 1))
    x_rows = xt.reshape(b * ch * t, n)
    tab_flat = time_table.T.reshape(NT * D)

    mesh = plsc.VectorSubcoreMesh(core_axis_name="c", subcore_axis_name="s")
    ebd_rows = pl.kernel(
        _sc_body,
        out_type=jax.ShapeDtypeStruct((b * t * D, n), jnp.float32),
        mesh=mesh,
        compiler_params=pltpu.CompilerParams(
            needs_layout_passes=False, use_tc_tiling_on_sc=True),
        scratch_types=[
            pltpu.VMEM((NT * D,), jnp.float32),
            pltpu.VMEM((G8, N), jnp.float32),
            pltpu.VMEM((D, N), jnp.float32),
            pltpu.VMEM((D, N), jnp.float32),
            pltpu.SemaphoreType.DMA,
            pltpu.SemaphoreType.DMA,
        ],
    )(x_rows, tab_flat)

    xo_t = pl.pallas_call(
        _tc_slice_body,
        grid=(b, NKEEP),
        in_specs=[pl.BlockSpec((1, 1, t, n), lambda i, c: (i, c, 0, 0))],
        out_specs=pl.BlockSpec((1, 1, t, n), lambda i, c: (i, c, 0, 0)),
        out_shape=jax.ShapeDtypeStruct((b, NKEEP, t, n), jnp.float32),
    )(xt)

    xo = jnp.transpose(xo_t, (0, 3, 2, 1))
    ebd = jnp.transpose(ebd_rows.reshape(b, t, D, n), (0, 3, 1, 2))
    return xo, ebd
